# baseline (speedup 1.0000x reference)
import functools

import jax
import jax.numpy as jnp
from jax import lax
from jax.experimental import pallas as pl
from jax.experimental.pallas import tpu as pltpu
from jax.experimental.pallas import tpu_sc as plsc

VOCAB = 1000000
DIM = 64
WDIM = 2 * DIM
BATCH = 4096
HIST = 50
TOTAL = BATCH * HIST

NC = 2
NS = 16
NW = NC * NS
SAMPLES_PER_W = BATCH // NW
IDS_PER_W = SAMPLES_PER_W * HIST
SAMPLES_PER_CHUNK = 2
CHUNK = SAMPLES_PER_CHUNK * HIST
NCHUNKS = IDS_PER_W // CHUNK
LANES = 16
CHUNK_PAD = 112
VPR = DIM // LANES


def _pool_body(values_hbm, wide_hbm, out_hbm, idx_v, rows0_v, rows1_v, acc_v,
               semg0, semg1):
    wid = lax.axis_index("s") * NC + lax.axis_index("c")

    pltpu.sync_copy(values_hbm.at[wid], idx_v)

    def _gather(c, rows, sem):
        return pltpu.async_copy(
            wide_hbm.at[idx_v.at[pl.ds(c * CHUNK_PAD, CHUNK)]], rows, sem)

    def _gather_wait(rows, sem):
        pltpu.make_async_copy(
            wide_hbm.at[idx_v.at[pl.ds(0, CHUNK)]], rows, sem).wait()

    def _accum(c, rows):
        for s in range(SAMPLES_PER_CHUNK):
            r0 = s * HIST
            a = [rows[r0, pl.ds(j * LANES, LANES)] for j in range(VPR)]
            b = [rows[r0 + 1, pl.ds(j * LANES, LANES)] for j in range(VPR)]
            for r in range(2, HIST, 2):
                for j in range(VPR):
                    a[j] = a[j] + rows[r0 + r, pl.ds(j * LANES, LANES)]
                    b[j] = b[j] + rows[r0 + r + 1, pl.ds(j * LANES, LANES)]
            smp = c * SAMPLES_PER_CHUNK + s
            for j in range(VPR):
                acc_v[smp, pl.ds(j * LANES, LANES)] = a[j] + b[j]

    _gather(0, rows0_v, semg0)
    _gather(1, rows1_v, semg1)

    def _pair(cp, _):
        c0 = 2 * cp
        _gather_wait(rows0_v, semg0)
        _accum(c0, rows0_v)
        _gather(c0 + 2, rows0_v, semg0)
        _gather_wait(rows1_v, semg1)
        _accum(c0 + 1, rows1_v)
        _gather(c0 + 3, rows1_v, semg1)
        return 0

    lax.fori_loop(0, (NCHUNKS - 2) // 2, _pair, 0)

    _gather_wait(rows0_v, semg0)
    _accum(NCHUNKS - 2, rows0_v)
    _gather_wait(rows1_v, semg1)
    _accum(NCHUNKS - 1, rows1_v)

    pltpu.sync_copy(acc_v, out_hbm.at[pl.ds(wid * SAMPLES_PER_W,
                                            SAMPLES_PER_W)])


@jax.jit
def _pooled_lookup(values, table):
    mesh = plsc.VectorSubcoreMesh(core_axis_name="c", subcore_axis_name="s")
    pool = functools.partial(
        pl.kernel,
        out_type=jax.ShapeDtypeStruct((BATCH, DIM), jnp.float32),
        mesh=mesh,
        compiler_params=pltpu.CompilerParams(use_tc_tiling_on_sc=True),
        scratch_types=[
            pltpu.VMEM((NCHUNKS * CHUNK_PAD,), jnp.int32),
            pltpu.VMEM((CHUNK, WDIM), jnp.float32),
            pltpu.VMEM((CHUNK, WDIM), jnp.float32),
            pltpu.VMEM((SAMPLES_PER_W, DIM), jnp.float32),
            pltpu.SemaphoreType.DMA,
            pltpu.SemaphoreType.DMA,
        ],
    )(_pool_body)
    one = (values[0] & 0).astype(jnp.float32) + 1.0
    wide = jnp.pad(table * one, ((0, 0), (0, WDIM - DIM)))
    vals = jnp.pad(values.reshape(NW, NCHUNKS, CHUNK),
                   ((0, 0), (0, 0), (0, CHUNK_PAD - CHUNK)))
    return pool(vals.reshape(NW, NCHUNKS * CHUNK_PAD), wide)


def kernel(values, lengths, table):
    del lengths
    return _pooled_lookup(values.astype(jnp.int32), table)


# speedup vs baseline: 1.0017x; 1.0017x over previous
import functools

import jax
import jax.numpy as jnp
from jax import lax
from jax.experimental import pallas as pl
from jax.experimental.pallas import tpu as pltpu
from jax.experimental.pallas import tpu_sc as plsc

VOCAB = 1000000
DIM = 64
WDIM = 2 * DIM
BATCH = 4096
HIST = 50
TOTAL = BATCH * HIST

NC = 2
NS = 16
NW = NC * NS
SAMPLES_PER_W = BATCH // NW
IDS_PER_W = SAMPLES_PER_W * HIST
SAMPLES_PER_CHUNK = 2
CHUNK = SAMPLES_PER_CHUNK * HIST
NCHUNKS = IDS_PER_W // CHUNK
LANES = 16
CHUNK_PAD = 112
VPR = DIM // LANES


def _pool_body(values_hbm, wide_hbm, out_hbm, idx_v, rows0_v, rows1_v, acc_v,
               semg0, semg1):
    wid = lax.axis_index("s") * NC + lax.axis_index("c")

    pltpu.sync_copy(values_hbm.at[wid], idx_v)

    def _gather(c, rows, sem):
        return pltpu.async_copy(
            wide_hbm.at[idx_v.at[pl.ds(c * CHUNK_PAD, CHUNK)]], rows, sem)

    def _gather_wait(rows, sem):
        pltpu.make_async_copy(
            wide_hbm.at[idx_v.at[pl.ds(0, CHUNK)]], rows, sem).wait()

    def _accum(c, rows):
        for s in range(SAMPLES_PER_CHUNK):
            r0 = s * HIST
            a = [rows[r0, pl.ds(j * LANES, LANES)] for j in range(VPR)]
            b = [rows[r0 + 1, pl.ds(j * LANES, LANES)] for j in range(VPR)]
            for r in range(2, HIST, 2):
                for j in range(VPR):
                    a[j] = a[j] + rows[r0 + r, pl.ds(j * LANES, LANES)]
                    b[j] = b[j] + rows[r0 + r + 1, pl.ds(j * LANES, LANES)]
            smp = c * SAMPLES_PER_CHUNK + s
            for j in range(VPR):
                acc_v[smp, pl.ds(j * LANES, LANES)] = a[j] + b[j]

    _gather(0, rows0_v, semg0)
    _gather(1, rows1_v, semg1)

    def _pair(cp, _):
        c0 = 2 * cp
        _gather_wait(rows0_v, semg0)
        _accum(c0, rows0_v)
        _gather(c0 + 2, rows0_v, semg0)
        _gather_wait(rows1_v, semg1)
        _accum(c0 + 1, rows1_v)
        _gather(c0 + 3, rows1_v, semg1)
        return 0

    lax.fori_loop(0, (NCHUNKS - 2) // 2, _pair, 0)

    _gather_wait(rows0_v, semg0)
    _accum(NCHUNKS - 2, rows0_v)
    _gather_wait(rows1_v, semg1)
    _accum(NCHUNKS - 1, rows1_v)

    pltpu.sync_copy(acc_v, out_hbm.at[pl.ds(wid * SAMPLES_PER_W,
                                            SAMPLES_PER_W)])


@jax.jit
def _pooled_lookup(values, table):
    mesh = plsc.VectorSubcoreMesh(core_axis_name="c", subcore_axis_name="s")
    pool = functools.partial(
        pl.kernel,
        out_type=jax.ShapeDtypeStruct((BATCH, DIM), jnp.float32),
        mesh=mesh,
        compiler_params=pltpu.CompilerParams(use_tc_tiling_on_sc=True),
        scratch_types=[
            pltpu.VMEM((NCHUNKS * CHUNK_PAD,), jnp.int32),
            pltpu.VMEM((CHUNK, WDIM), jnp.float32),
            pltpu.VMEM((CHUNK, WDIM), jnp.float32),
            pltpu.VMEM((SAMPLES_PER_W, DIM), jnp.float32),
            pltpu.SemaphoreType.DMA,
            pltpu.SemaphoreType.DMA,
        ],
    )(_pool_body)
    one = (values[0] // (values[0] + 1)).astype(jnp.float32) + 1.0
    wide = jnp.pad(table, ((0, 0), (0, WDIM - DIM))) * one
    vals = jnp.pad(values.reshape(NW, NCHUNKS, CHUNK),
                   ((0, 0), (0, 0), (0, CHUNK_PAD - CHUNK)))
    return pool(vals.reshape(NW, NCHUNKS * CHUNK_PAD), wide)


def kernel(values, lengths, table):
    del lengths
    return _pooled_lookup(values.astype(jnp.int32), table)
